# baseline (speedup 1.0000x reference)
import functools

import jax
import jax.numpy as jnp
from jax import lax
from jax.experimental import pallas as pl
from jax.experimental.pallas import tpu as pltpu
from jax.experimental.pallas import tpu_sc as plsc

VOCAB = 1000000
D = 32
B = 4096
L = 200
W = 128

NC = 2
NS = 16
NW = NC * NS
B_PER_W = B // NW
IDX_PER_W = B_PER_W * L

C0 = 96
C1 = 104


def _body(xq_hbm, xr_hbm, tw_hbm, out_hbm, xq_v, xr_v, rows_a, rows_b, out_v,
          sem0, sem1):
    wid = lax.axis_index("s") * NC + lax.axis_index("c")
    base = wid * IDX_PER_W
    pltpu.sync_copy(xq_hbm.at[pl.ds(base, IDX_PER_W)], xq_v)
    pltpu.sync_copy(xr_hbm.at[pl.ds(base, IDX_PER_W)], xr_v.at[pl.ds(0, IDX_PER_W)])

    def copies(b, buf, sem):
        off = b * L
        c0 = pltpu.make_async_copy(
            tw_hbm.at[xq_v.at[pl.ds(off, C0)]], buf.at[pl.ds(0, C0)], sem)
        c1 = pltpu.make_async_copy(
            tw_hbm.at[xq_v.at[pl.ds(off + C0, C1)]], buf.at[pl.ds(C0, C1)], sem)
        return c0, c1

    def fire(b, buf, sem):
        c0, c1 = copies(b, buf, sem)
        c0.start()
        c1.start()

    def drain(b, buf, sem):
        c0, c1 = copies(b, buf, sem)
        c0.wait()
        c1.wait()

    def pool_row(b, buf):
        rowbase = b * L

        def gbody(g, accs):
            a0, a1, a2, a3 = accs
            off_v = xr_v[pl.ds(rowbase + g * 16, 16)]
            jb = g * 16
            for j in range(0, 16, 2):
                w0 = off_v[j]
                w1 = off_v[j + 1]
                a0 = a0 + buf[jb + j, pl.ds(w0, 16)]
                a1 = a1 + buf[jb + j, pl.ds(w0 + 16, 16)]
                a2 = a2 + buf[jb + j + 1, pl.ds(w1, 16)]
                a3 = a3 + buf[jb + j + 1, pl.ds(w1 + 16, 16)]
            return a0, a1, a2, a3

        z = jnp.zeros((16,), jnp.float32)
        a0, a1, a2, a3 = lax.fori_loop(0, L // 16, gbody, (z, z, z, z))
        off_v = xr_v[pl.ds(rowbase + (L // 16) * 16, 16)]
        for j in range(0, L - (L // 16) * 16, 2):
            jb = (L // 16) * 16
            w0 = off_v[j]
            w1 = off_v[j + 1]
            a0 = a0 + buf[jb + j, pl.ds(w0, 16)]
            a1 = a1 + buf[jb + j, pl.ds(w0 + 16, 16)]
            a2 = a2 + buf[jb + j + 1, pl.ds(w1, 16)]
            a3 = a3 + buf[jb + j + 1, pl.ds(w1 + 16, 16)]
        m0 = (a0 + a2) * jnp.float32(1.0 / L)
        m1 = (a1 + a3) * jnp.float32(1.0 / L)
        ss = plsc.cumsum(m0 * m0 + m1 * m1)[15]
        i = lax.bitcast_convert_type(ss, jnp.int32)
        i = jnp.int32(0x5F3759DF) - lax.shift_right_logical(i, 1)
        y = lax.bitcast_convert_type(i, jnp.float32)
        for _ in range(3):
            y = y * (jnp.float32(1.5) - jnp.float32(0.5) * ss * y * y)
        d = jnp.maximum(ss * y, jnp.float32(1e-12))
        wr = b // 4
        wc = (b % 4) * 32
        out_v[wr, pl.ds(wc, 16)] = m0 / d
        out_v[wr, pl.ds(wc + 16, 16)] = m1 / d

    fire(0, rows_a, sem0)
    fire(1, rows_b, sem1)

    def step(g, carry):
        b0 = 2 * g
        b1 = b0 + 1
        drain(b0, rows_a, sem0)
        pool_row(b0, rows_a)

        @pl.when(b0 + 2 < B_PER_W)
        def _():
            fire(b0 + 2, rows_a, sem0)

        drain(b1, rows_b, sem1)
        pool_row(b1, rows_b)

        @pl.when(b1 + 2 < B_PER_W)
        def _():
            fire(b1 + 2, rows_b, sem1)

        return carry

    lax.fori_loop(0, B_PER_W // 2, step, 0)
    pltpu.sync_copy(out_v, out_hbm.at[pl.ds(wid * (B_PER_W * D // W), B_PER_W * D // W)])


_embed_pool = functools.partial(
    pl.kernel,
    out_type=jax.ShapeDtypeStruct((B * D // W, W), jnp.float32),
    mesh=plsc.VectorSubcoreMesh(
        core_axis_name="c", subcore_axis_name="s", num_cores=NC, num_subcores=NS),
    compiler_params=pltpu.CompilerParams(
        needs_layout_passes=False, use_tc_tiling_on_sc=True),
    scratch_types=[
        pltpu.VMEM((IDX_PER_W,), jnp.int32),
        pltpu.VMEM((IDX_PER_W + 16,), jnp.int32),
        pltpu.VMEM((L, W), jnp.float32),
        pltpu.VMEM((L, W), jnp.float32),
        pltpu.VMEM((B_PER_W * D // W, W), jnp.float32),
        pltpu.SemaphoreType.DMA,
        pltpu.SemaphoreType.DMA,
    ],
)(_body)


def kernel(x, table):
    xi = jnp.reshape(x.astype(jnp.int32), (B * L,))
    xq = xi >> 2
    xr = (xi & 3) << 5
    tw = jnp.reshape(table, (VOCAB * D // W, W))
    ow = _embed_pool(xq, xr, tw)
    return jnp.reshape(ow, (B, D))


# speedup vs baseline: 1.1428x; 1.1428x over previous
import functools

import jax
import jax.numpy as jnp
from jax import lax
from jax.experimental import pallas as pl
from jax.experimental.pallas import tpu as pltpu
from jax.experimental.pallas import tpu_sc as plsc

D = 32
B = 4096
L = 200

NC = 2
NS = 16
NW = NC * NS
B_PER_W = B // NW
IDX_PER_W = B_PER_W * L

C0 = 96
C1 = 104


def _body(idx_hbm, table_hbm, out_hbm, idx_v, rows_a, rows_b, out_v, sem0, sem1):
    wid = lax.axis_index("s") * NC + lax.axis_index("c")
    base = wid * IDX_PER_W
    pltpu.sync_copy(idx_hbm.at[pl.ds(base, IDX_PER_W)], idx_v)

    def copies(b, buf, sem):
        off = b * L
        c0 = pltpu.make_async_copy(
            table_hbm.at[idx_v.at[pl.ds(off, C0)]], buf.at[pl.ds(0, C0)], sem)
        c1 = pltpu.make_async_copy(
            table_hbm.at[idx_v.at[pl.ds(off + C0, C1)]], buf.at[pl.ds(C0, C1)], sem)
        return c0, c1

    def fire(b, buf, sem):
        c0, c1 = copies(b, buf, sem)
        c0.start()
        c1.start()

    def drain(b, buf, sem):
        c0, c1 = copies(b, buf, sem)
        c0.wait()
        c1.wait()

    def pool_row(b, buf):
        def rbody(j, accs):
            a0, a1, a2, a3 = accs
            a0 = a0 + buf[2 * j, pl.ds(0, 16)]
            a1 = a1 + buf[2 * j, pl.ds(16, 16)]
            a2 = a2 + buf[2 * j + 1, pl.ds(0, 16)]
            a3 = a3 + buf[2 * j + 1, pl.ds(16, 16)]
            return a0, a1, a2, a3

        z = jnp.zeros((16,), jnp.float32)
        a0, a1, a2, a3 = lax.fori_loop(0, L // 2, rbody, (z, z, z, z), unroll=4)
        m0 = (a0 + a2) * jnp.float32(1.0 / L)
        m1 = (a1 + a3) * jnp.float32(1.0 / L)
        ss = plsc.cumsum(m0 * m0 + m1 * m1)[15]
        i = lax.bitcast_convert_type(ss, jnp.int32)
        i = jnp.int32(0x5F3759DF) - lax.shift_right_logical(i, 1)
        y = lax.bitcast_convert_type(i, jnp.float32)
        for _ in range(3):
            y = y * (jnp.float32(1.5) - jnp.float32(0.5) * ss * y * y)
        d = jnp.maximum(ss * y, jnp.float32(1e-12))
        out_v[b, pl.ds(0, 16)] = m0 / d
        out_v[b, pl.ds(16, 16)] = m1 / d

    fire(0, rows_a, sem0)
    fire(1, rows_b, sem1)

    def step(g, carry):
        b0 = 2 * g
        b1 = b0 + 1
        drain(b0, rows_a, sem0)
        pool_row(b0, rows_a)

        @pl.when(b0 + 2 < B_PER_W)
        def _():
            fire(b0 + 2, rows_a, sem0)

        drain(b1, rows_b, sem1)
        pool_row(b1, rows_b)

        @pl.when(b1 + 2 < B_PER_W)
        def _():
            fire(b1 + 2, rows_b, sem1)

        return carry

    lax.fori_loop(0, B_PER_W // 2, step, 0)
    pltpu.sync_copy(out_v, out_hbm.at[pl.ds(wid * B_PER_W, B_PER_W)])


_embed_pool = functools.partial(
    pl.kernel,
    out_type=jax.ShapeDtypeStruct((B, D), jnp.float32),
    mesh=plsc.VectorSubcoreMesh(
        core_axis_name="c", subcore_axis_name="s", num_cores=NC, num_subcores=NS),
    compiler_params=pltpu.CompilerParams(
        needs_layout_passes=False, use_tc_tiling_on_sc=False),
    scratch_types=[
        pltpu.VMEM((IDX_PER_W,), jnp.int32),
        pltpu.VMEM((L, D), jnp.float32),
        pltpu.VMEM((L, D), jnp.float32),
        pltpu.VMEM((B_PER_W, D), jnp.float32),
        pltpu.SemaphoreType.DMA,
        pltpu.SemaphoreType.DMA,
    ],
)(_body)


VOCAB = 1000000
RB = 8000


QB = RB // 4


def _compact_body(t_ref, o_ref):
    t = t_ref[...]
    w = jnp.concatenate([t[k * QB:(k + 1) * QB] for k in range(4)], axis=1)
    o_ref[...] = jnp.reshape(w, (RB * D,))


_compact = pl.pallas_call(
    _compact_body,
    grid=(VOCAB // RB,),
    in_specs=[pl.BlockSpec((RB, D), lambda i: (i, 0))],
    out_specs=pl.BlockSpec((RB * D,), lambda i: (i,)),
    out_shape=jax.ShapeDtypeStruct((VOCAB * D,), jnp.float32),
)


def kernel(x, table):
    xi = jnp.reshape(x.astype(jnp.int32), (B * L,))
    u = xi % RB
    xf = (xi - u) + (u % QB) * 4 + u // QB
    tlin = jnp.reshape(_compact(table), (VOCAB, D))
    return _embed_pool(xf, tlin)
